# Initial kernel scaffold; baseline (speedup 1.0000x reference)
import jax
import jax.numpy as jnp
from jax.experimental import pallas as pl


def kernel(x, gate_w, Wg, Wu, Wd, Sg, Su, Sd, shared_scale):
    raise NotImplementedError("write your pallas kernel here")


# speedup vs baseline: 2.2977x; 2.2977x over previous
import functools

import jax
import jax.numpy as jnp
from jax import lax
from jax.experimental import pallas as pl
from jax.experimental.pallas import tpu as pltpu
from jax.experimental.pallas import tpu_sc as plsc

_T, _C, _H = 4096, 768, 768
_E = 64
_E1 = _E + 1
_K = 2
_P = _K * _T
_P3 = _P + _T
_B = 128
_NB = _P // _B + _E + _T // _B
_NP = _NB * _B

_NC, _NS = 2, 16
_NW = _NC * _NS

_DSUB = 64
_DCH = _NP // _NW
_NDS = _DCH // _DSUB

_TPW = _T // _NW
_CSUB = 32
_NCS = _TPW // _CSUB


def _gate_body(x_ref, gw_ref, ti_ref, tw_ref):
    xb = x_ref[...]
    logits = lax.dot_general(xb, gw_ref[...], (((1,), (1,)), ((), ())),
                             preferred_element_type=jnp.float32)
    iota = lax.broadcasted_iota(jnp.int32, logits.shape, 1)
    m1 = jnp.max(logits, axis=1, keepdims=True)
    i1 = jnp.min(jnp.where(logits == m1, iota, _E), axis=1, keepdims=True)
    l2 = jnp.where(iota == i1, -jnp.inf, logits)
    m2 = jnp.max(l2, axis=1, keepdims=True)
    i2 = jnp.min(jnp.where(l2 == m2, iota, _E), axis=1, keepdims=True)
    w1 = jax.nn.sigmoid(m1 - m2)
    ti_ref[...] = jnp.concatenate([i1, i2], axis=1)
    tw_ref[...] = jnp.concatenate([w1, 1.0 - w1], axis=1)


def _gate(x, gate_w):
    rb = 1024
    return pl.pallas_call(
        _gate_body,
        grid=(_T // rb,),
        in_specs=[
            pl.BlockSpec((rb, _C), lambda i: (i, 0)),
            pl.BlockSpec((_E, _C), lambda i: (0, 0)),
        ],
        out_specs=[
            pl.BlockSpec((rb, _K), lambda i: (i, 0)),
            pl.BlockSpec((rb, _K), lambda i: (i, 0)),
        ],
        out_shape=[
            jax.ShapeDtypeStruct((_T, _K), jnp.int32),
            jax.ShapeDtypeStruct((_T, _K), jnp.float32),
        ],
    )(x, gate_w)


def _moe_body(be_ref, nb_ref, xs_ref, wg_ref, wu_ref, wd_ref,
              sg_ref, su_ref, sd_ref, ws_ref, os_ref):
    i = pl.program_id(0)
    e = be_ref[i]
    live = i < nb_ref[0]

    def compute(wg, wu, wd):
        xb = xs_ref[...]
        g = jnp.dot(xb, wg, preferred_element_type=jnp.float32)
        u = jnp.dot(xb, wu, preferred_element_type=jnp.float32)
        h = g * jax.nn.sigmoid(g) * u
        ob = jnp.dot(h, wd, preferred_element_type=jnp.float32)
        w = ws_ref[...]
        return ob * jnp.concatenate([w] * (_C // 128), axis=1)

    @pl.when(jnp.logical_and(live, e < _E))
    def _():
        os_ref[...] = compute(wg_ref[0], wu_ref[0], wd_ref[0])

    @pl.when(jnp.logical_and(live, e == _E))
    def _():
        os_ref[...] = compute(sg_ref[...], su_ref[...], sd_ref[...])


def _moe(be, nb, xs, Wg, Wu, Wd, Sg, Su, Sd, ws_b):
    grid_spec = pltpu.PrefetchScalarGridSpec(
        num_scalar_prefetch=2,
        grid=(_NB,),
        in_specs=[
            pl.BlockSpec((_B, _C), lambda i, be, nb: (i, 0)),
            pl.BlockSpec((1, _C, _H),
                         lambda i, be, nb: (jnp.minimum(be[i], _E - 1), 0, 0)),
            pl.BlockSpec((1, _C, _H),
                         lambda i, be, nb: (jnp.minimum(be[i], _E - 1), 0, 0)),
            pl.BlockSpec((1, _H, _C),
                         lambda i, be, nb: (jnp.minimum(be[i], _E - 1), 0, 0)),
            pl.BlockSpec((_C, _H), lambda i, be, nb: (0, 0)),
            pl.BlockSpec((_C, _H), lambda i, be, nb: (0, 0)),
            pl.BlockSpec((_H, _C), lambda i, be, nb: (0, 0)),
            pl.BlockSpec((_B, 128), lambda i, be, nb: (i, 0)),
        ],
        out_specs=pl.BlockSpec((_B, _C), lambda i, be, nb: (i, 0)),
    )
    return pl.pallas_call(
        _moe_body,
        grid_spec=grid_spec,
        out_shape=jax.ShapeDtypeStruct((_NP, _C), jnp.float32),
    )(be, nb, xs, Wg, Wu, Wd, Sg, Su, Sd, ws_b)


def _sc_dispatch(x, src):
    mesh = plsc.VectorSubcoreMesh(core_axis_name="c", subcore_axis_name="s")

    @functools.partial(
        pl.kernel,
        out_type=jax.ShapeDtypeStruct((_NP, _C), jnp.float32),
        mesh=mesh,
        scratch_types=[
            pltpu.VMEM((_DSUB,), jnp.int32),
            pltpu.VMEM((_DSUB, _C), jnp.float32),
            pltpu.SemaphoreType.DMA,
        ],
    )
    def k(x_hbm, src_hbm, xs_hbm, idx_v, rows_v, sem):
        wid = lax.axis_index("s") * _NC + lax.axis_index("c")
        base = wid * _DCH

        def step(s, carry):
            off = base + s * _DSUB
            pltpu.sync_copy(src_hbm.at[pl.ds(off, _DSUB)], idx_v)
            pltpu.async_copy(x_hbm.at[idx_v], rows_v, sem).wait()
            pltpu.sync_copy(rows_v, xs_hbm.at[pl.ds(off, _DSUB)])
            return carry

        lax.fori_loop(0, _NDS, step, 0)

    return k(x, src)


def _sc_combine(os_, pos0, pos1, pos2):
    mesh = plsc.VectorSubcoreMesh(core_axis_name="c", subcore_axis_name="s")

    @functools.partial(
        pl.kernel,
        out_type=jax.ShapeDtypeStruct((_T, _C), jnp.float32),
        mesh=mesh,
        scratch_types=[
            pltpu.VMEM((_CSUB,), jnp.int32),
            pltpu.VMEM((_CSUB,), jnp.int32),
            pltpu.VMEM((_CSUB,), jnp.int32),
            pltpu.VMEM((_CSUB, _C), jnp.float32),
            pltpu.VMEM((_CSUB, _C), jnp.float32),
            pltpu.VMEM((_CSUB, _C), jnp.float32),
            pltpu.SemaphoreType.DMA,
        ],
    )
    def k(os_hbm, p0_hbm, p1_hbm, p2_hbm, y_hbm, i0, i1, i2, a, b, c, sem):
        wid = lax.axis_index("s") * _NC + lax.axis_index("c")
        base = wid * _TPW

        def step(s, carry):
            off = base + s * _CSUB
            pltpu.sync_copy(p0_hbm.at[pl.ds(off, _CSUB)], i0)
            pltpu.sync_copy(p1_hbm.at[pl.ds(off, _CSUB)], i1)
            pltpu.sync_copy(p2_hbm.at[pl.ds(off, _CSUB)], i2)
            pltpu.async_copy(os_hbm.at[i0], a, sem).wait()
            pltpu.async_copy(os_hbm.at[i1], b, sem).wait()
            pltpu.async_copy(os_hbm.at[i2], c, sem).wait()

            def row(j, carry2):
                for kk in range(_C // 16):
                    sl = pl.ds(kk * 16, 16)
                    c[j, sl] = a[j, sl] + b[j, sl] + c[j, sl]
                return carry2

            lax.fori_loop(0, _CSUB, row, 0)
            pltpu.sync_copy(c, y_hbm.at[pl.ds(off, _CSUB)])
            return carry

        lax.fori_loop(0, _NCS, step, 0)

    return k(os_, pos0, pos1, pos2)


def kernel(x, gate_w, Wg, Wu, Wd, Sg, Su, Sd, shared_scale):
    ti, tw = _gate(x, gate_w)

    e3 = jnp.concatenate([ti.reshape(_P),
                          jnp.full((_T,), _E, jnp.int32)])
    tw3 = jnp.concatenate([tw.reshape(_P),
                           jnp.broadcast_to(shared_scale.astype(jnp.float32),
                                            (_T,))])
    tok3 = jnp.concatenate([
        (jnp.arange(_P, dtype=jnp.int32) // _K),
        jnp.arange(_T, dtype=jnp.int32),
    ])

    counts = jnp.bincount(e3, length=_E1).astype(jnp.int32)
    order = jnp.argsort(e3).astype(jnp.int32)
    pc = ((counts + _B - 1) // _B) * _B
    pend = jnp.cumsum(pc)
    poff = pend - pc
    start = jnp.cumsum(counts) - counts
    e_s = e3[order]
    jj = jnp.arange(_P3, dtype=jnp.int32)
    ppos = (poff[e_s] + (jj - start[e_s])).astype(jnp.int32)

    src = jnp.zeros((_NP,), jnp.int32).at[ppos].set(tok3[order])
    ws = jnp.zeros((_NP,), jnp.float32).at[ppos].set(tw3[order])
    pos_pair = jnp.zeros((_P3,), jnp.int32).at[order].set(ppos)
    pos01 = pos_pair[:_P].reshape(_T, _K)
    pos0 = pos01[:, 0]
    pos1 = pos01[:, 1]
    pos2 = pos_pair[_P:]

    total = pend[-1]
    nb = (total // _B).astype(jnp.int32).reshape(1)
    bs = jnp.arange(_NB, dtype=jnp.int32) * _B
    be = jnp.searchsorted(pend, bs, side="right").astype(jnp.int32)
    be = jnp.where(bs < total, be, _E)
    ws_b = jnp.broadcast_to(ws[:, None], (_NP, 128))

    xs = _sc_dispatch(x, src)
    os_ = _moe(be, nb, xs, Wg, Wu, Wd, Sg, Su, Sd, ws_b)
    y = _sc_combine(os_, pos0, pos1, pos2)
    return y


# speedup vs baseline: 3.5372x; 1.5395x over previous
import functools

import jax
import jax.numpy as jnp
from jax import lax
from jax.experimental import pallas as pl
from jax.experimental.pallas import tpu as pltpu
from jax.experimental.pallas import tpu_sc as plsc

_T, _C, _H = 4096, 768, 768
_E = 64
_E1 = _E + 1
_K = 2
_P = _K * _T
_P3 = _P + _T
_B = 128
_NBR = _P // _B + _E
_NPR = _NBR * _B
_NB = _NBR + _T // _B
_NP = _NB * _B

_NC, _NS = 2, 16
_NW = _NC * _NS

_DSUB = 64
_DCH = _P // _NW
_NDS = _DCH // _DSUB

_TPW = _T // _NW
_CSUB = 32
_NCS = _TPW // _CSUB


def _gate_body(x_ref, gw_ref, ti_ref, tw_ref):
    xb = x_ref[...]
    logits = lax.dot_general(xb, gw_ref[...], (((1,), (1,)), ((), ())),
                             preferred_element_type=jnp.float32)
    iota = lax.broadcasted_iota(jnp.int32, logits.shape, 1)
    m1 = jnp.max(logits, axis=1, keepdims=True)
    i1 = jnp.min(jnp.where(logits == m1, iota, _E), axis=1, keepdims=True)
    l2 = jnp.where(iota == i1, -jnp.inf, logits)
    m2 = jnp.max(l2, axis=1, keepdims=True)
    i2 = jnp.min(jnp.where(l2 == m2, iota, _E), axis=1, keepdims=True)
    w1 = jax.nn.sigmoid(m1 - m2)
    ti_ref[...] = jnp.concatenate([i1, i2], axis=1)
    tw_ref[...] = jnp.concatenate([w1, 1.0 - w1], axis=1)


def _gate(x, gate_w):
    rb = 1024
    return pl.pallas_call(
        _gate_body,
        grid=(_T // rb,),
        in_specs=[
            pl.BlockSpec((rb, _C), lambda i: (i, 0)),
            pl.BlockSpec((_E, _C), lambda i: (0, 0)),
        ],
        out_specs=[
            pl.BlockSpec((rb, _K), lambda i: (i, 0)),
            pl.BlockSpec((rb, _K), lambda i: (i, 0)),
        ],
        out_shape=[
            jax.ShapeDtypeStruct((_T, _K), jnp.int32),
            jax.ShapeDtypeStruct((_T, _K), jnp.float32),
        ],
    )(x, gate_w)


def _moe_body(be_ref, nb_ref, sc_ref, xs_ref, x_ref, wg_ref, wu_ref, wd_ref,
              sg_ref, su_ref, sd_ref, ws_ref, os_ref):
    i = pl.program_id(0)
    e = be_ref[i]
    live = i < nb_ref[0]

    def compute(xb, wg, wu, wd):
        g = jnp.dot(xb, wg, preferred_element_type=jnp.float32)
        u = jnp.dot(xb, wu, preferred_element_type=jnp.float32)
        h = g * jax.nn.sigmoid(g) * u
        return jnp.dot(h, wd, preferred_element_type=jnp.float32)

    @pl.when(jnp.logical_and(live, e < _E))
    def _():
        ob = compute(xs_ref[...], wg_ref[0], wu_ref[0], wd_ref[0])
        w = ws_ref[...]
        os_ref[...] = ob * jnp.concatenate([w] * (_C // 128), axis=1)

    @pl.when(jnp.logical_and(live, e == _E))
    def _():
        ob = compute(x_ref[...], sg_ref[...], su_ref[...], sd_ref[...])
        os_ref[...] = ob * sc_ref[0]


def _moe(be, nb, scale, xs, x, Wg, Wu, Wd, Sg, Su, Sd, ws_b):
    grid_spec = pltpu.PrefetchScalarGridSpec(
        num_scalar_prefetch=3,
        grid=(_NB,),
        in_specs=[
            pl.BlockSpec((_B, _C),
                         lambda i, be, nb, sc: (jnp.minimum(i, nb[0] - 33), 0)),
            pl.BlockSpec((_B, _C),
                         lambda i, be, nb, sc: (
                             jnp.clip(i - (nb[0] - _T // _B), 0,
                                      _T // _B - 1), 0)),
            pl.BlockSpec((1, _C, _H),
                         lambda i, be, nb, sc: (jnp.minimum(be[i], _E - 1), 0, 0)),
            pl.BlockSpec((1, _C, _H),
                         lambda i, be, nb, sc: (jnp.minimum(be[i], _E - 1), 0, 0)),
            pl.BlockSpec((1, _H, _C),
                         lambda i, be, nb, sc: (jnp.minimum(be[i], _E - 1), 0, 0)),
            pl.BlockSpec((_C, _H), lambda i, be, nb, sc: (0, 0)),
            pl.BlockSpec((_C, _H), lambda i, be, nb, sc: (0, 0)),
            pl.BlockSpec((_H, _C), lambda i, be, nb, sc: (0, 0)),
            pl.BlockSpec((_B, 128),
                         lambda i, be, nb, sc: (jnp.minimum(i, nb[0] - 33), 0)),
        ],
        out_specs=pl.BlockSpec((_B, _C), lambda i, be, nb, sc: (i, 0)),
    )
    return pl.pallas_call(
        _moe_body,
        grid_spec=grid_spec,
        out_shape=jax.ShapeDtypeStruct((_NP, _C), jnp.float32),
    )(be, nb, scale, xs, x, Wg, Wu, Wd, Sg, Su, Sd, ws_b)


def _sc_dispatch(x, tok2, pp2):
    mesh = plsc.VectorSubcoreMesh(core_axis_name="c", subcore_axis_name="s")

    @functools.partial(
        pl.kernel,
        out_type=jax.ShapeDtypeStruct((_NPR, _C), jnp.float32),
        mesh=mesh,
        scratch_types=[
            pltpu.VMEM((_NDS, _DSUB), jnp.int32),
            pltpu.VMEM((_NDS, _DSUB), jnp.int32),
            pltpu.VMEM((_DSUB, _C), jnp.float32),
            pltpu.VMEM((_DSUB, _C), jnp.float32),
            pltpu.SemaphoreType.DMA,
            pltpu.SemaphoreType.DMA,
            pltpu.SemaphoreType.DMA,
        ],
    )
    def k(x_hbm, tok_hbm, pp_hbm, xs_hbm, tok_v, pp_v, r0, r1, gsem,
          ssem0, ssem1):
        wid = lax.axis_index("s") * _NC + lax.axis_index("c")
        pltpu.sync_copy(tok_hbm.at[pl.ds(wid * _NDS, _NDS)], tok_v)
        pltpu.sync_copy(pp_hbm.at[pl.ds(wid * _NDS, _NDS)], pp_v)
        rows = (r0, r1)
        ssems = (ssem0, ssem1)
        scatters = []
        for s in range(_NDS):
            buf = s % 2
            if s >= 2:
                scatters[s - 2].wait()
            pltpu.async_copy(x_hbm.at[tok_v.at[s]], rows[buf], gsem).wait()
            scatters.append(
                pltpu.async_copy(rows[buf], xs_hbm.at[pp_v.at[s]],
                                 ssems[buf]))
        scatters[_NDS - 2].wait()
        scatters[_NDS - 1].wait()

    return k(x, tok2, pp2)


def _sc_combine(os_, pos0, pos1, pos2):
    mesh = plsc.VectorSubcoreMesh(core_axis_name="c", subcore_axis_name="s")

    @functools.partial(
        pl.kernel,
        out_type=jax.ShapeDtypeStruct((_T, _C), jnp.float32),
        mesh=mesh,
        scratch_types=[
            pltpu.VMEM((_CSUB,), jnp.int32),
            pltpu.VMEM((_CSUB,), jnp.int32),
            pltpu.VMEM((_CSUB,), jnp.int32),
            pltpu.VMEM((_CSUB, _C), jnp.float32),
            pltpu.VMEM((_CSUB, _C), jnp.float32),
            pltpu.VMEM((_CSUB, _C), jnp.float32),
            pltpu.SemaphoreType.DMA,
        ],
    )
    def k(os_hbm, p0_hbm, p1_hbm, p2_hbm, y_hbm, i0, i1, i2, a, b, c, sem):
        wid = lax.axis_index("s") * _NC + lax.axis_index("c")
        base = wid * _TPW

        def step(s, carry):
            off = base + s * _CSUB
            pltpu.sync_copy(p0_hbm.at[pl.ds(off, _CSUB)], i0)
            pltpu.sync_copy(p1_hbm.at[pl.ds(off, _CSUB)], i1)
            pltpu.sync_copy(p2_hbm.at[pl.ds(off, _CSUB)], i2)
            pltpu.async_copy(os_hbm.at[i0], a, sem).wait()
            pltpu.async_copy(os_hbm.at[i1], b, sem).wait()
            pltpu.async_copy(os_hbm.at[i2], c, sem).wait()

            def row(j, carry2):
                for kk in range(_C // 16):
                    sl = pl.ds(kk * 16, 16)
                    c[j, sl] = a[j, sl] + b[j, sl] + c[j, sl]
                return carry2

            lax.fori_loop(0, _CSUB, row, 0)
            pltpu.sync_copy(c, y_hbm.at[pl.ds(off, _CSUB)])
            return carry

        lax.fori_loop(0, _NCS, step, 0)

    return k(os_, pos0, pos1, pos2)


def kernel(x, gate_w, Wg, Wu, Wd, Sg, Su, Sd, shared_scale):
    ti, tw = _gate(x, gate_w)

    e3 = jnp.concatenate([ti.reshape(_P),
                          jnp.full((_T,), _E, jnp.int32)])
    tw3 = jnp.concatenate([tw.reshape(_P),
                           jnp.broadcast_to(shared_scale.astype(jnp.float32),
                                            (_T,))])
    tok3 = jnp.concatenate([
        (jnp.arange(_P, dtype=jnp.int32) // _K),
        jnp.arange(_T, dtype=jnp.int32),
    ])

    counts = jnp.bincount(e3, length=_E1).astype(jnp.int32)
    order = jnp.argsort(e3).astype(jnp.int32)
    pc = ((counts + _B - 1) // _B) * _B
    pend = jnp.cumsum(pc)
    poff = pend - pc
    start = jnp.cumsum(counts) - counts
    e_s = e3[order]
    jj = jnp.arange(_P3, dtype=jnp.int32)
    ppos = (poff[e_s] + (jj - start[e_s])).astype(jnp.int32)

    ws = jnp.zeros((_NPR,), jnp.float32).at[ppos].set(tw3[order])
    pos_pair = jnp.zeros((_P3,), jnp.int32).at[order].set(ppos)
    pos01 = pos_pair[:_P].reshape(_T, _K)
    pos0 = pos01[:, 0]
    pos1 = pos01[:, 1]
    pos2 = pos_pair[_P:]
    tok2 = tok3[order][:_P].reshape(_NW * _NDS, _DSUB)
    pp2 = ppos[:_P].reshape(_NW * _NDS, _DSUB)

    total = pend[-1]
    nb = (total // _B).astype(jnp.int32).reshape(1)
    bs = jnp.arange(_NB, dtype=jnp.int32) * _B
    be = jnp.searchsorted(pend, bs, side="right").astype(jnp.int32)
    be = jnp.where(bs < total, be, _E)
    ws_b = jnp.broadcast_to(ws[:, None], (_NPR, 128))
    scale = shared_scale.astype(jnp.float32).reshape(1)

    xs = _sc_dispatch(x, tok2, pp2)
    os_ = _moe(be, nb, scale, xs, x, Wg, Wu, Wd, Sg, Su, Sd, ws_b)
    y = _sc_combine(os_, pos0, pos1, pos2)
    return y


# speedup vs baseline: 5.1315x; 1.4507x over previous
import functools

import jax
import jax.numpy as jnp
from jax import lax
from jax.experimental import pallas as pl
from jax.experimental.pallas import tpu as pltpu
from jax.experimental.pallas import tpu_sc as plsc

_T, _C, _H = 4096, 768, 768
_E = 64
_E1 = _E + 1
_K = 2
_P = _K * _T
_P3 = _P + _T
_B = 128
_NBR = _P // _B + _E
_NPR = _NBR * _B
_NB = _NBR + _T // _B
_NP = _NB * _B

_NC, _NS = 2, 16
_NW = _NC * _NS

_DSUB = 64
_DCH = _P // _NW
_NDS = _DCH // _DSUB

_TPW = _T // _NW
_CSUB = 32
_NCS = _TPW // _CSUB


def _gate_body(x_ref, gw_ref, ti_ref, tw_ref):
    xb = x_ref[...]
    logits = lax.dot_general(xb, gw_ref[...], (((1,), (1,)), ((), ())),
                             preferred_element_type=jnp.float32)
    iota = lax.broadcasted_iota(jnp.int32, logits.shape, 1)
    m1 = jnp.max(logits, axis=1, keepdims=True)
    i1 = jnp.min(jnp.where(logits == m1, iota, _E), axis=1, keepdims=True)
    l2 = jnp.where(iota == i1, -jnp.inf, logits)
    m2 = jnp.max(l2, axis=1, keepdims=True)
    i2 = jnp.min(jnp.where(l2 == m2, iota, _E), axis=1, keepdims=True)
    w1 = jax.nn.sigmoid(m1 - m2)
    ti_ref[...] = jnp.concatenate([i1, i2], axis=1)
    tw_ref[...] = jnp.concatenate([w1, 1.0 - w1], axis=1)


def _gate(x, gate_w):
    rb = 1024
    return pl.pallas_call(
        _gate_body,
        grid=(_T // rb,),
        in_specs=[
            pl.BlockSpec((rb, _C), lambda i: (i, 0)),
            pl.BlockSpec((_E, _C), lambda i: (0, 0)),
        ],
        out_specs=[
            pl.BlockSpec((rb, _K), lambda i: (i, 0)),
            pl.BlockSpec((rb, _K), lambda i: (i, 0)),
        ],
        out_shape=[
            jax.ShapeDtypeStruct((_T, _K), jnp.int32),
            jax.ShapeDtypeStruct((_T, _K), jnp.float32),
        ],
    )(x, gate_w)


def _moe_body(be_ref, nb_ref, sc_ref, xs_ref, x_ref, wg_ref, wu_ref, wd_ref,
              sg_ref, su_ref, sd_ref, ws_ref, os_ref):
    i = pl.program_id(0)
    e = be_ref[i]
    live = i < nb_ref[0]

    def compute(xb, wg, wu, wd):
        g = jnp.dot(xb, wg, preferred_element_type=jnp.float32)
        u = jnp.dot(xb, wu, preferred_element_type=jnp.float32)
        h = g * jax.nn.sigmoid(g) * u
        return jnp.dot(h, wd, preferred_element_type=jnp.float32)

    @pl.when(jnp.logical_and(live, e < _E))
    def _():
        ob = compute(xs_ref[...], wg_ref[0], wu_ref[0], wd_ref[0])
        w = ws_ref[...]
        os_ref[...] = ob * jnp.concatenate([w] * (_C // 128), axis=1)

    @pl.when(jnp.logical_and(live, e == _E))
    def _():
        ob = compute(x_ref[...], sg_ref[...], su_ref[...], sd_ref[...])
        os_ref[...] = ob * sc_ref[0]


def _moe(be, nb, scale, xs, x, Wg, Wu, Wd, Sg, Su, Sd, ws_b):
    grid_spec = pltpu.PrefetchScalarGridSpec(
        num_scalar_prefetch=3,
        grid=(_NB,),
        in_specs=[
            pl.BlockSpec((_B, _C),
                         lambda i, be, nb, sc: (jnp.minimum(i, nb[0] - 33), 0)),
            pl.BlockSpec((_B, _C),
                         lambda i, be, nb, sc: (
                             jnp.clip(i - (nb[0] - _T // _B), 0,
                                      _T // _B - 1), 0)),
            pl.BlockSpec((1, _C, _H),
                         lambda i, be, nb, sc: (jnp.minimum(be[i], _E - 1), 0, 0)),
            pl.BlockSpec((1, _C, _H),
                         lambda i, be, nb, sc: (jnp.minimum(be[i], _E - 1), 0, 0)),
            pl.BlockSpec((1, _H, _C),
                         lambda i, be, nb, sc: (jnp.minimum(be[i], _E - 1), 0, 0)),
            pl.BlockSpec((_C, _H), lambda i, be, nb, sc: (0, 0)),
            pl.BlockSpec((_C, _H), lambda i, be, nb, sc: (0, 0)),
            pl.BlockSpec((_H, _C), lambda i, be, nb, sc: (0, 0)),
            pl.BlockSpec((_B, 128),
                         lambda i, be, nb, sc: (jnp.minimum(i, nb[0] - 33), 0)),
        ],
        out_specs=pl.BlockSpec((_B, _C), lambda i, be, nb, sc: (i, 0)),
    )
    return pl.pallas_call(
        _moe_body,
        grid_spec=grid_spec,
        out_shape=jax.ShapeDtypeStruct((_NP, _C), jnp.float32),
    )(be, nb, scale, xs, x, Wg, Wu, Wd, Sg, Su, Sd, ws_b)


def _sc_dispatch(x, tok2, pp2):
    mesh = plsc.VectorSubcoreMesh(core_axis_name="c", subcore_axis_name="s")

    @functools.partial(
        pl.kernel,
        out_type=jax.ShapeDtypeStruct((_NPR, _C), jnp.float32),
        mesh=mesh,
        scratch_types=[
            pltpu.VMEM((_NDS, _DSUB), jnp.int32),
            pltpu.VMEM((_NDS, _DSUB), jnp.int32),
            pltpu.VMEM((_DSUB, _C), jnp.float32),
            pltpu.VMEM((_DSUB, _C), jnp.float32),
            pltpu.SemaphoreType.DMA,
            pltpu.SemaphoreType.DMA,
            pltpu.SemaphoreType.DMA,
        ],
    )
    def k(x_hbm, tok_hbm, pp_hbm, xs_hbm, tok_v, pp_v, r0, r1, gsem,
          ssem0, ssem1):
        wid = lax.axis_index("s") * _NC + lax.axis_index("c")
        pltpu.sync_copy(tok_hbm.at[pl.ds(wid * _NDS, _NDS)], tok_v)
        pltpu.sync_copy(pp_hbm.at[pl.ds(wid * _NDS, _NDS)], pp_v)
        rows = (r0, r1)
        ssems = (ssem0, ssem1)
        scatters = []
        for s in range(_NDS):
            buf = s % 2
            if s >= 2:
                scatters[s - 2].wait()
            pltpu.async_copy(x_hbm.at[tok_v.at[s]], rows[buf], gsem).wait()
            scatters.append(
                pltpu.async_copy(rows[buf], xs_hbm.at[pp_v.at[s]],
                                 ssems[buf]))
        scatters[_NDS - 2].wait()
        scatters[_NDS - 1].wait()

    return k(x, tok2, pp2)


def _sc_combine(os_, pos0, pos1, pos2):
    mesh = plsc.VectorSubcoreMesh(core_axis_name="c", subcore_axis_name="s")

    @functools.partial(
        pl.kernel,
        out_type=jax.ShapeDtypeStruct((_T, _C), jnp.float32),
        mesh=mesh,
        scratch_types=[
            pltpu.VMEM((_CSUB,), jnp.int32),
            pltpu.VMEM((_CSUB,), jnp.int32),
            pltpu.VMEM((_CSUB,), jnp.int32),
            pltpu.VMEM((_CSUB, _C), jnp.float32),
            pltpu.VMEM((_CSUB, _C), jnp.float32),
            pltpu.VMEM((_CSUB, _C), jnp.float32),
            pltpu.SemaphoreType.DMA,
        ],
    )
    def k(os_hbm, p0_hbm, p1_hbm, p2_hbm, y_hbm, i0, i1, i2, a, b, c, sem):
        wid = lax.axis_index("s") * _NC + lax.axis_index("c")
        base = wid * _TPW

        def step(s, carry):
            off = base + s * _CSUB
            pltpu.sync_copy(p0_hbm.at[pl.ds(off, _CSUB)], i0)
            pltpu.sync_copy(p1_hbm.at[pl.ds(off, _CSUB)], i1)
            pltpu.sync_copy(p2_hbm.at[pl.ds(off, _CSUB)], i2)
            pltpu.async_copy(os_hbm.at[i0], a, sem).wait()
            pltpu.async_copy(os_hbm.at[i1], b, sem).wait()
            pltpu.async_copy(os_hbm.at[i2], c, sem).wait()

            def row(j, carry2):
                for kk in range(_C // 16):
                    sl = pl.ds(kk * 16, 16)
                    c[j, sl] = a[j, sl] + b[j, sl] + c[j, sl]
                return carry2

            lax.fori_loop(0, _CSUB, row, 0)
            pltpu.sync_copy(c, y_hbm.at[pl.ds(off, _CSUB)])
            return carry

        lax.fori_loop(0, _NCS, step, 0)

    return k(os_, pos0, pos1, pos2)


def kernel(x, gate_w, Wg, Wu, Wd, Sg, Su, Sd, shared_scale):
    ti, tw = _gate(x, gate_w)

    e3 = jnp.concatenate([ti.reshape(_P),
                          jnp.full((_T,), _E, jnp.int32)])
    tw3 = jnp.concatenate([tw.reshape(_P),
                           jnp.broadcast_to(shared_scale.astype(jnp.float32),
                                            (_T,))])
    tok3 = jnp.concatenate([
        (jnp.arange(_P, dtype=jnp.int32) // _K),
        jnp.arange(_T, dtype=jnp.int32),
    ])

    oh = (e3[:, None] == jnp.arange(_E1, dtype=jnp.int32)[None, :]).astype(
        jnp.int32)
    cum = jnp.cumsum(oh, axis=0)
    counts = cum[-1]
    pc = ((counts + _B - 1) // _B) * _B
    pend = jnp.cumsum(pc)
    poff = pend - pc
    rank = jnp.sum(cum * oh, axis=1) - 1
    ppos = (jnp.sum(oh * poff[None, :], axis=1) + rank).astype(jnp.int32)

    ws = jnp.zeros((_NPR,), jnp.float32).at[ppos].set(tw3, mode="drop")
    pos01 = ppos[:_P].reshape(_T, _K)
    pos0 = pos01[:, 0]
    pos1 = pos01[:, 1]
    pos2 = ppos[_P:]
    tok2 = tok3[:_P].reshape(_NW * _NDS, _DSUB)
    pp2 = ppos[:_P].reshape(_NW * _NDS, _DSUB)

    total = pend[-1]
    nb = (total // _B).astype(jnp.int32).reshape(1)
    bs = jnp.arange(_NB, dtype=jnp.int32) * _B
    be = jnp.searchsorted(pend, bs, side="right").astype(jnp.int32)
    be = jnp.where(bs < total, be, _E)
    ws_b = jnp.broadcast_to(ws[:, None], (_NPR, 128))
    scale = shared_scale.astype(jnp.float32).reshape(1)

    xs = _sc_dispatch(x, tok2, pp2)
    os_ = _moe(be, nb, scale, xs, x, Wg, Wu, Wd, Sg, Su, Sd, ws_b)
    y = _sc_combine(os_, pos0, pos1, pos2)
    return y
